# baseline (speedup 1.0000x reference)
import jax
import jax.numpy as jnp
from jax.experimental import pallas as pl
from jax.experimental.pallas import tpu as pltpu

_S = 2048
_F = _S // 2
_W = 256
_HPB = 4
_FB = 256
_TWO_PI_OVER_S = 2.0 * 3.14159265358979323846 / _S


def _rot(x, j):
    return jnp.concatenate([x[j:], x[:j]], axis=0)


def _split(x):
    hi = x.astype(jnp.bfloat16)
    lo = (x - hi.astype(jnp.float32)).astype(jnp.bfloat16)
    return hi, lo


def _mm3(ah, al, bh, bl):
    f32 = jnp.float32
    return (jnp.dot(ah, bh, preferred_element_type=f32)
            + jnp.dot(ah, bl, preferred_element_type=f32)
            + jnp.dot(al, bh, preferred_element_type=f32))


def _mm3t(ah, al, bh, bl):
    dn = (((0,), (0,)), ((), ()))
    f32 = jnp.float32
    dg = jax.lax.dot_general
    return (dg(ah, bh, dn, preferred_element_type=f32)
            + dg(ah, bl, dn, preferred_element_type=f32)
            + dg(al, bh, dn, preferred_element_type=f32))


def _bitonic_desc_inplace(scr):
    i = jax.lax.broadcasted_iota(jnp.int32, (_S, 1), 0)

    def outer(logk, carry):
        k = jnp.left_shift(jnp.int32(1), logk)
        for logj in range(10, -1, -1):
            j = 1 << logj

            @pl.when(j <= jax.lax.shift_right_logical(k, 1))
            def _stage():
                if j >= 8:
                    nb = _S // (2 * j)
                    x = scr[...].reshape(nb, 2, j, scr.shape[1])
                    a = x[:, 0]
                    b = x[:, 1]
                    mn = jnp.minimum(a, b)
                    mx = jnp.maximum(a, b)
                    m_i = jax.lax.broadcasted_iota(jnp.int32, (nb, 1, 1), 0)
                    desc = ((m_i * (2 * j)) & k) == 0
                    low = jnp.where(desc, mx, mn)[:, None]
                    high = jnp.where(desc, mn, mx)[:, None]
                    y = jnp.concatenate([low, high], axis=1)
                    scr[...] = y.reshape(_S, scr.shape[1])
                else:
                    x = scr[...]
                    upper = (i & j) != 0
                    partner = jnp.where(upper, _rot(x, _S - j), _rot(x, j))
                    want_max = ((i & j) == 0) == ((i & k) == 0)
                    scr[...] = jnp.where(want_max, jnp.maximum(x, partner),
                                         jnp.minimum(x, partner))
        return carry

    jax.lax.fori_loop(1, 12, outer, 0)


def _body(q_ref, k_ref, v_ref, o_ref,
          ch_scr, cl_scr, sh_scr, sl_scr,
          reh_scr, rel_scr, imh_scr, iml_scr,
          p_scr, pb_scr):
    @pl.when(pl.program_id(0) == 0)
    def _init():
        def fwd_mat_blk(fb, carry):
            row0 = fb * _FB
            f = jax.lax.broadcasted_iota(jnp.int32, (_FB, _S), 0) + row0
            n = jax.lax.broadcasted_iota(jnp.int32, (_FB, _S), 1)
            m = (f * n) & (_S - 1)
            ang = m.astype(jnp.float32) * jnp.float32(_TWO_PI_OVER_S)
            ch, cl = _split(jnp.cos(ang))
            sh, sl = _split(jnp.sin(ang))
            ch_scr[pl.ds(row0, _FB), :] = ch
            cl_scr[pl.ds(row0, _FB), :] = cl
            sh_scr[pl.ds(row0, _FB), :] = sh
            sl_scr[pl.ds(row0, _FB), :] = sl
            return carry
        jax.lax.fori_loop(0, _F // _FB, fwd_mat_blk, 0)

    seq_i = jax.lax.broadcasted_iota(jnp.int32, (_S, 1), 0)
    parity = jnp.where((seq_i & 1) == 0, jnp.float32(1), jnp.float32(-1))
    qn = jnp.sum(q_ref[...] * parity, axis=0, keepdims=True)
    kn = jnp.sum(k_ref[...] * parity, axis=0, keepdims=True)
    ren = qn * kn

    qh, ql = _split(q_ref[...])
    kh, kl = _split(k_ref[...])

    def fwd_blk(fb, carry):
        row0 = fb * _FB
        ch = ch_scr[pl.ds(row0, _FB), :]
        cl = cl_scr[pl.ds(row0, _FB), :]
        sh = sh_scr[pl.ds(row0, _FB), :]
        sl = sl_scr[pl.ds(row0, _FB), :]
        qc = _mm3(ch, cl, qh, ql)
        qs = _mm3(sh, sl, qh, ql)
        kc = _mm3(ch, cl, kh, kl)
        ks = _mm3(sh, sl, kh, kl)
        reh, rel = _split(qc * kc + qs * ks)
        imh, iml = _split(qc * ks - qs * kc)
        reh_scr[pl.ds(row0, _FB), :] = reh
        rel_scr[pl.ds(row0, _FB), :] = rel
        imh_scr[pl.ds(row0, _FB), :] = imh
        iml_scr[pl.ds(row0, _FB), :] = iml
        return carry
    jax.lax.fori_loop(0, _F // _FB, fwd_blk, 0)

    re0 = (reh_scr[0:1, :].astype(jnp.float32)
           + rel_scr[0:1, :].astype(jnp.float32))

    blk_i = jax.lax.broadcasted_iota(jnp.int32, (_FB, 1), 0)
    par_fb = jnp.where((blk_i & 1) == 0, jnp.float32(1), jnp.float32(-1))

    def inv_blk(nb, carry):
        row0 = nb * _FB
        cih = ch_scr[:, pl.ds(row0, _FB)]
        cil = cl_scr[:, pl.ds(row0, _FB)]
        sih = sh_scr[:, pl.ds(row0, _FB)]
        sil = sl_scr[:, pl.ds(row0, _FB)]
        acc = (_mm3t(cih, cil, reh_scr[...], rel_scr[...])
               - _mm3t(sih, sil, imh_scr[...], iml_scr[...])) * jnp.float32(2.0 / _S)
        corr = acc + (par_fb * ren - re0) * jnp.float32(1.0 / _S)
        p_scr[pl.ds(row0, _FB), :] = corr
        return carry
    jax.lax.fori_loop(0, _S // _FB, inv_blk, 0)

    corr = p_scr[...]
    mx = jnp.max(corr, axis=0, keepdims=True)
    e = jnp.exp(corr - mx)
    pb_scr[...] = (e / jnp.sum(e, axis=0, keepdims=True)).astype(jnp.bfloat16)

    _bitonic_desc_inplace(pb_scr)
    psh = pb_scr[...]

    v = v_ref[...]
    for h in range(_HPB):
        sl = slice(h * 64, (h + 1) * 64)
        vh, vl = _split(v[h])
        f32 = jnp.float32
        o_ref[:, sl] = (jnp.dot(psh[:, sl], vh, preferred_element_type=f32)
                        + jnp.dot(psh[:, sl], vl, preferred_element_type=f32))


def kernel(queries, keys, values):
    B, H, S, dk = queries.shape
    q2 = jnp.transpose(queries[0], (1, 0, 2)).reshape(S, H * dk)
    k2 = jnp.transpose(keys[0], (1, 0, 2)).reshape(S, H * dk)
    v3 = values[0, :, :dk, :]

    nblk = (H * dk) // _W
    f32 = jnp.float32
    bf16 = jnp.bfloat16
    out2 = pl.pallas_call(
        _body,
        grid=(nblk,),
        in_specs=[
            pl.BlockSpec((S, _W), lambda b: (0, b)),
            pl.BlockSpec((S, _W), lambda b: (0, b)),
            pl.BlockSpec((_HPB, dk, dk), lambda b: (b, 0, 0)),
        ],
        out_specs=pl.BlockSpec((S, _W), lambda b: (0, b)),
        out_shape=jax.ShapeDtypeStruct((S, H * dk), f32),
        scratch_shapes=[
            pltpu.VMEM((_F, _S), bf16),
            pltpu.VMEM((_F, _S), bf16),
            pltpu.VMEM((_F, _S), bf16),
            pltpu.VMEM((_F, _S), bf16),
            pltpu.VMEM((_F, _W), bf16),
            pltpu.VMEM((_F, _W), bf16),
            pltpu.VMEM((_F, _W), bf16),
            pltpu.VMEM((_F, _W), bf16),
            pltpu.VMEM((_S, _W), f32),
            pltpu.VMEM((_S, _W), bf16),
        ],
    )(q2, k2, v3)

    return out2.reshape(S, H, dk).transpose(1, 0, 2)[None]


# speedup vs baseline: 1.0147x; 1.0147x over previous
import jax
import jax.numpy as jnp
from jax.experimental import pallas as pl
from jax.experimental.pallas import tpu as pltpu

_S = 2048
_F = _S // 2
_W = 256
_HPB = 4
_FB = 256
_TWO_PI_OVER_S = 2.0 * 3.14159265358979323846 / _S


def _rot(x, j):
    return jnp.concatenate([x[j:], x[:j]], axis=0)


def _split(x):
    hi = x.astype(jnp.bfloat16)
    lo = (x - hi.astype(jnp.float32)).astype(jnp.bfloat16)
    return hi, lo


def _mm3(ah, al, bh, bl):
    f32 = jnp.float32
    return (jnp.dot(ah, bh, preferred_element_type=f32)
            + jnp.dot(ah, bl, preferred_element_type=f32)
            + jnp.dot(al, bh, preferred_element_type=f32))


def _mm3t(ah, al, bh, bl):
    dn = (((0,), (0,)), ((), ()))
    f32 = jnp.float32
    dg = jax.lax.dot_general
    return (dg(ah, bh, dn, preferred_element_type=f32)
            + dg(ah, bl, dn, preferred_element_type=f32)
            + dg(al, bh, dn, preferred_element_type=f32))


def _bitonic_desc_inplace(scr):
    i = jax.lax.broadcasted_iota(jnp.int32, (_S, 1), 0)

    def outer(logk, carry):
        k = jnp.left_shift(jnp.int32(1), logk)
        for logj in range(10, -1, -1):
            j = 1 << logj

            @pl.when(j <= jax.lax.shift_right_logical(k, 1))
            def _stage():
                if j >= 8:
                    nb = _S // (2 * j)
                    x = scr[...].reshape(nb, 2, j, scr.shape[1])
                    a = x[:, 0]
                    b = x[:, 1]
                    mn = jnp.minimum(a, b)
                    mx = jnp.maximum(a, b)
                    m_i = jax.lax.broadcasted_iota(jnp.int32, (nb, 1, 1), 0)
                    desc = ((m_i * (2 * j)) & k) == 0
                    low = jnp.where(desc, mx, mn)[:, None]
                    high = jnp.where(desc, mn, mx)[:, None]
                    y = jnp.concatenate([low, high], axis=1)
                    scr[...] = y.reshape(_S, scr.shape[1])
                else:
                    x = scr[...]
                    upper = (i & j) != 0
                    partner = jnp.where(upper, _rot(x, _S - j), _rot(x, j))
                    want_max = ((i & j) == 0) == ((i & k) == 0)
                    scr[...] = jnp.where(want_max, jnp.maximum(x, partner),
                                         jnp.minimum(x, partner))
        return carry

    jax.lax.fori_loop(1, 12, outer, 0)


def _body(q_ref, k_ref, v_ref, o_ref,
          ch_scr, cl_scr, sh_scr, sl_scr,
          reh_scr, rel_scr, imh_scr, iml_scr,
          p_scr, pb_scr):
    @pl.when(pl.program_id(0) == 0)
    def _init():
        def fwd_mat_blk(fb, carry):
            row0 = fb * _FB
            f = jax.lax.broadcasted_iota(jnp.int32, (_FB, _S), 0) + row0
            n = jax.lax.broadcasted_iota(jnp.int32, (_FB, _S), 1)
            m = (f * n) & (_S - 1)
            ang = m.astype(jnp.float32) * jnp.float32(_TWO_PI_OVER_S)
            ch, cl = _split(jnp.cos(ang))
            sh, sl = _split(jnp.sin(ang))
            ch_scr[pl.ds(row0, _FB), :] = ch
            cl_scr[pl.ds(row0, _FB), :] = cl
            sh_scr[pl.ds(row0, _FB), :] = sh
            sl_scr[pl.ds(row0, _FB), :] = sl
            return carry
        jax.lax.fori_loop(0, _F // _FB, fwd_mat_blk, 0)

    seq_i = jax.lax.broadcasted_iota(jnp.int32, (_S, 1), 0)
    parity = jnp.where((seq_i & 1) == 0, jnp.float32(1), jnp.float32(-1))
    qn = jnp.sum(q_ref[...] * parity, axis=0, keepdims=True)
    kn = jnp.sum(k_ref[...] * parity, axis=0, keepdims=True)
    ren = qn * kn

    def fwd_blk(fb, carry):
        row0 = fb * _FB
        ch = ch_scr[pl.ds(row0, _FB), :]
        cl = cl_scr[pl.ds(row0, _FB), :]
        sh = sh_scr[pl.ds(row0, _FB), :]
        sl = sl_scr[pl.ds(row0, _FB), :]
        qh, ql = _split(q_ref[...])
        kh, kl = _split(k_ref[...])
        qc = _mm3(ch, cl, qh, ql)
        qs = _mm3(sh, sl, qh, ql)
        kc = _mm3(ch, cl, kh, kl)
        ks = _mm3(sh, sl, kh, kl)
        reh, rel = _split(qc * kc + qs * ks)
        imh, iml = _split(qc * ks - qs * kc)
        reh_scr[pl.ds(row0, _FB), :] = reh
        rel_scr[pl.ds(row0, _FB), :] = rel
        imh_scr[pl.ds(row0, _FB), :] = imh
        iml_scr[pl.ds(row0, _FB), :] = iml
        return carry
    jax.lax.fori_loop(0, _F // _FB, fwd_blk, 0)

    re0 = (reh_scr[0:1, :].astype(jnp.float32)
           + rel_scr[0:1, :].astype(jnp.float32))

    blk_i = jax.lax.broadcasted_iota(jnp.int32, (_FB, 1), 0)
    par_fb = jnp.where((blk_i & 1) == 0, jnp.float32(1), jnp.float32(-1))

    def inv_blk(nb, carry):
        row0 = nb * _FB
        cih = ch_scr[:, pl.ds(row0, _FB)]
        cil = cl_scr[:, pl.ds(row0, _FB)]
        sih = sh_scr[:, pl.ds(row0, _FB)]
        sil = sl_scr[:, pl.ds(row0, _FB)]
        acc = (_mm3t(cih, cil, reh_scr[...], rel_scr[...])
               - _mm3t(sih, sil, imh_scr[...], iml_scr[...])) * jnp.float32(2.0 / _S)
        corr = acc + (par_fb * ren - re0) * jnp.float32(1.0 / _S)
        p_scr[pl.ds(row0, _FB), :] = corr
        return carry
    jax.lax.fori_loop(0, _S // _FB, inv_blk, 0)

    corr = p_scr[...]
    mx = jnp.max(corr, axis=0, keepdims=True)
    e = jnp.exp(corr - mx)
    pb_scr[...] = (e / jnp.sum(e, axis=0, keepdims=True)).astype(jnp.bfloat16)

    _bitonic_desc_inplace(pb_scr)
    psh = pb_scr[...]

    v = v_ref[...]
    for h in range(_HPB):
        sl = slice(h * 64, (h + 1) * 64)
        vh, vl = _split(v[h])
        f32 = jnp.float32
        o_ref[:, sl] = (jnp.dot(psh[:, sl], vh, preferred_element_type=f32)
                        + jnp.dot(psh[:, sl], vl, preferred_element_type=f32))


def kernel(queries, keys, values):
    B, H, S, dk = queries.shape
    q2 = jnp.transpose(queries[0], (1, 0, 2)).reshape(S, H * dk)
    k2 = jnp.transpose(keys[0], (1, 0, 2)).reshape(S, H * dk)
    v3 = values[0, :, :dk, :]

    nblk = (H * dk) // _W
    f32 = jnp.float32
    bf16 = jnp.bfloat16
    out2 = pl.pallas_call(
        _body,
        grid=(nblk,),
        in_specs=[
            pl.BlockSpec((S, _W), lambda b: (0, b)),
            pl.BlockSpec((S, _W), lambda b: (0, b)),
            pl.BlockSpec((_HPB, dk, dk), lambda b: (b, 0, 0)),
        ],
        out_specs=pl.BlockSpec((S, _W), lambda b: (0, b)),
        out_shape=jax.ShapeDtypeStruct((S, H * dk), f32),
        scratch_shapes=[
            pltpu.VMEM((_F, _S), bf16),
            pltpu.VMEM((_F, _S), bf16),
            pltpu.VMEM((_F, _S), bf16),
            pltpu.VMEM((_F, _S), bf16),
            pltpu.VMEM((_F, _W), bf16),
            pltpu.VMEM((_F, _W), bf16),
            pltpu.VMEM((_F, _W), bf16),
            pltpu.VMEM((_F, _W), bf16),
            pltpu.VMEM((_S, _W), f32),
            pltpu.VMEM((_S, _W), bf16),
        ],
    )(q2, k2, v3)

    return out2.reshape(S, H, dk).transpose(1, 0, 2)[None]


# speedup vs baseline: 1.1258x; 1.1095x over previous
import jax
import jax.numpy as jnp
from jax.experimental import pallas as pl
from jax.experimental.pallas import tpu as pltpu

_S = 2048
_F = _S // 2
_W = 256
_HPB = 4
_FB = 256
_TWO_PI_OVER_S = 2.0 * 3.14159265358979323846 / _S


def _rot(x, j):
    return jnp.concatenate([x[j:], x[:j]], axis=0)


def _split(x):
    hi = x.astype(jnp.bfloat16)
    lo = (x - hi.astype(jnp.float32)).astype(jnp.bfloat16)
    return hi, lo


def _mm3(ah, al, bh, bl):
    f32 = jnp.float32
    return (jnp.dot(ah, bh, preferred_element_type=f32)
            + jnp.dot(ah, bl, preferred_element_type=f32)
            + jnp.dot(al, bh, preferred_element_type=f32))


def _mm3t(ah, al, bh, bl):
    dn = (((0,), (0,)), ((), ()))
    f32 = jnp.float32
    dg = jax.lax.dot_general
    return (dg(ah, bh, dn, preferred_element_type=f32)
            + dg(ah, bl, dn, preferred_element_type=f32)
            + dg(al, bh, dn, preferred_element_type=f32))


def _bitonic_desc_inplace(scr):
    i = jax.lax.broadcasted_iota(jnp.int32, (_S, 1), 0)

    def outer(logk, carry):
        k = jnp.left_shift(jnp.int32(1), logk)
        for logj in range(10, -1, -1):
            j = 1 << logj

            @pl.when(j <= jax.lax.shift_right_logical(k, 1))
            def _stage():
                if j >= 8:
                    nb = _S // (2 * j)
                    x = scr[...].reshape(nb, 2, j, scr.shape[1])
                    a = x[:, 0]
                    b = x[:, 1]
                    mn = jnp.minimum(a, b)
                    mx = jnp.maximum(a, b)
                    m_i = jax.lax.broadcasted_iota(jnp.int32, (nb, 1, 1), 0)
                    desc = ((m_i * (2 * j)) & k) == 0
                    low = jnp.where(desc, mx, mn)[:, None]
                    high = jnp.where(desc, mn, mx)[:, None]
                    y = jnp.concatenate([low, high], axis=1)
                    scr[...] = y.reshape(_S, scr.shape[1])
                else:
                    x = scr[...]
                    upper = (i & j) != 0
                    partner = jnp.where(upper, _rot(x, _S - j), _rot(x, j))
                    want_max = ((i & j) == 0) == ((i & k) == 0)
                    scr[...] = jnp.where(want_max, jnp.maximum(x, partner),
                                         jnp.minimum(x, partner))
        return carry

    jax.lax.fori_loop(1, 12, outer, 0)


def _body(q_ref, k_ref, v_ref, o_ref,
          ch_scr, cl_scr, sh_scr, sl_scr,
          reh_scr, rel_scr, imh_scr, iml_scr,
          p_scr, pb_scr):
    @pl.when(pl.program_id(0) == 0)
    def _init():
        f_i = jax.lax.broadcasted_iota(jnp.int32, (_FB, _S), 0)
        n_i = jax.lax.broadcasted_iota(jnp.int32, (_FB, _S), 1)
        m0 = (f_i * n_i) & (_S - 1)
        ang0 = m0.astype(jnp.float32) * jnp.float32(_TWO_PI_OVER_S)
        c0 = jnp.cos(ang0)
        s0 = jnp.sin(ang0)
        ch, cl = _split(c0)
        sh, sl = _split(s0)
        ch_scr[pl.ds(0, _FB), :] = ch
        cl_scr[pl.ds(0, _FB), :] = cl
        sh_scr[pl.ds(0, _FB), :] = sh
        sl_scr[pl.ds(0, _FB), :] = sl
        n_row = jax.lax.broadcasted_iota(jnp.int32, (1, _S), 1)
        for b in range(1, _F // _FB):
            mb = (n_row * (_FB * b)) & (_S - 1)
            angb = mb.astype(jnp.float32) * jnp.float32(_TWO_PI_OVER_S)
            cb = jnp.cos(angb)
            sb = jnp.sin(angb)
            cbl = c0 * cb - s0 * sb
            sbl = s0 * cb + c0 * sb
            ch, cl = _split(cbl)
            sh, sl = _split(sbl)
            row0 = _FB * b
            ch_scr[pl.ds(row0, _FB), :] = ch
            cl_scr[pl.ds(row0, _FB), :] = cl
            sh_scr[pl.ds(row0, _FB), :] = sh
            sl_scr[pl.ds(row0, _FB), :] = sl

    seq_i = jax.lax.broadcasted_iota(jnp.int32, (_S, 1), 0)
    parity = jnp.where((seq_i & 1) == 0, jnp.float32(1), jnp.float32(-1))
    qn = jnp.sum(q_ref[...] * parity, axis=0, keepdims=True)
    kn = jnp.sum(k_ref[...] * parity, axis=0, keepdims=True)
    ren = qn * kn

    def fwd_blk(fb, carry):
        row0 = fb * _FB
        ch = ch_scr[pl.ds(row0, _FB), :]
        cl = cl_scr[pl.ds(row0, _FB), :]
        sh = sh_scr[pl.ds(row0, _FB), :]
        sl = sl_scr[pl.ds(row0, _FB), :]
        qh, ql = _split(q_ref[...])
        kh, kl = _split(k_ref[...])
        qc = _mm3(ch, cl, qh, ql)
        qs = _mm3(sh, sl, qh, ql)
        kc = _mm3(ch, cl, kh, kl)
        ks = _mm3(sh, sl, kh, kl)
        reh, rel = _split(qc * kc + qs * ks)
        imh, iml = _split(qc * ks - qs * kc)
        reh_scr[pl.ds(row0, _FB), :] = reh
        rel_scr[pl.ds(row0, _FB), :] = rel
        imh_scr[pl.ds(row0, _FB), :] = imh
        iml_scr[pl.ds(row0, _FB), :] = iml
        return carry
    jax.lax.fori_loop(0, _F // _FB, fwd_blk, 0)

    re0 = (reh_scr[0:1, :].astype(jnp.float32)
           + rel_scr[0:1, :].astype(jnp.float32))

    blk_i = jax.lax.broadcasted_iota(jnp.int32, (_FB, 1), 0)
    par_fb = jnp.where((blk_i & 1) == 0, jnp.float32(1), jnp.float32(-1))

    def inv_blk(nb, carry):
        row0 = nb * _FB
        cih = ch_scr[:, pl.ds(row0, _FB)]
        cil = cl_scr[:, pl.ds(row0, _FB)]
        sih = sh_scr[:, pl.ds(row0, _FB)]
        sil = sl_scr[:, pl.ds(row0, _FB)]
        acc = (_mm3t(cih, cil, reh_scr[...], rel_scr[...])
               - _mm3t(sih, sil, imh_scr[...], iml_scr[...])) * jnp.float32(2.0 / _S)
        corr = acc + (par_fb * ren - re0) * jnp.float32(1.0 / _S)
        p_scr[pl.ds(row0, _FB), :] = corr
        return carry
    jax.lax.fori_loop(0, _S // _FB, inv_blk, 0)

    corr = p_scr[...]
    mx = jnp.max(corr, axis=0, keepdims=True)
    e = jnp.exp(corr - mx)
    pb_scr[...] = (e / jnp.sum(e, axis=0, keepdims=True)).astype(jnp.bfloat16)

    _bitonic_desc_inplace(pb_scr)
    psh = pb_scr[...]

    v = v_ref[...]
    for h in range(_HPB):
        sl = slice(h * 64, (h + 1) * 64)
        vh, vl = _split(v[h])
        f32 = jnp.float32
        o_ref[:, sl] = (jnp.dot(psh[:, sl], vh, preferred_element_type=f32)
                        + jnp.dot(psh[:, sl], vl, preferred_element_type=f32))


def kernel(queries, keys, values):
    B, H, S, dk = queries.shape
    q2 = jnp.transpose(queries[0], (1, 0, 2)).reshape(S, H * dk)
    k2 = jnp.transpose(keys[0], (1, 0, 2)).reshape(S, H * dk)
    v3 = values[0, :, :dk, :]

    nblk = (H * dk) // _W
    f32 = jnp.float32
    bf16 = jnp.bfloat16
    out2 = pl.pallas_call(
        _body,
        grid=(nblk,),
        in_specs=[
            pl.BlockSpec((S, _W), lambda b: (0, b)),
            pl.BlockSpec((S, _W), lambda b: (0, b)),
            pl.BlockSpec((_HPB, dk, dk), lambda b: (b, 0, 0)),
        ],
        out_specs=pl.BlockSpec((S, _W), lambda b: (0, b)),
        out_shape=jax.ShapeDtypeStruct((S, H * dk), f32),
        scratch_shapes=[
            pltpu.VMEM((_F, _S), bf16),
            pltpu.VMEM((_F, _S), bf16),
            pltpu.VMEM((_F, _S), bf16),
            pltpu.VMEM((_F, _S), bf16),
            pltpu.VMEM((_F, _W), bf16),
            pltpu.VMEM((_F, _W), bf16),
            pltpu.VMEM((_F, _W), bf16),
            pltpu.VMEM((_F, _W), bf16),
            pltpu.VMEM((_S, _W), f32),
            pltpu.VMEM((_S, _W), bf16),
        ],
    )(q2, k2, v3)

    return out2.reshape(S, H, dk).transpose(1, 0, 2)[None]
